# baseline (speedup 1.0000x reference)
import jax
import jax.numpy as jnp
from jax import lax
from jax.experimental import pallas as pl
from jax.experimental.pallas import tpu as pltpu
from jax.experimental.pallas import tpu_sc as plsc

B = 4096
NUM_ID = 26
SEQ_LEN = 50
VOCAB = 100000
ED = 32
NW = 32
RPW = B // NW
PW_BAG = RPW * SEQ_LEN
CH_BAG = 800
NCH = PW_BAG // CH_BAG


TRR = 4000


def _tr_body(x_r, o_r):
    x4 = x_r[0].reshape(TRR // 4, 4, ED)
    o_r[...] = jnp.concatenate(
        [x4[:, 0], x4[:, 1], x4[:, 2], x4[:, 3]],
        axis=1).reshape(1, TRR // 4, 128)


def _transpose_table(emb_id):
    return pl.pallas_call(
        _tr_body,
        grid=(NUM_ID, VOCAB // TRR),
        in_specs=[pl.BlockSpec((1, TRR, ED), lambda f, j: (f, j, 0))],
        out_specs=pl.BlockSpec((1, TRR // 4, 128), lambda f, j: (f, j, 0)),
        out_shape=jax.ShapeDtypeStruct((NUM_ID, VOCAB // 4, 128), jnp.float32),
    )(emb_id)


def _sc_id_body(emb128_h, q4_h, o_field4, idx_v, buf):
    wid = lax.axis_index("s") * 2 + lax.axis_index("c")
    rb = wid * RPW

    @pl.loop(0, NUM_ID)
    def _(f):
        pltpu.sync_copy(q4_h.at[pl.ds(f * B + rb, RPW)], idx_v.at[pl.ds(0, RPW)])
        pltpu.sync_copy(emb128_h.at[f].at[idx_v.at[pl.ds(0, RPW)]],
                        buf.at[pl.ds(0, RPW)])
        pltpu.sync_copy(buf.at[pl.ds(0, RPW)],
                        o_field4.at[pl.ds(rb, RPW), pl.ds(f * 128, 128)])


def _sc_id_gather(emb128, q4_flat):
    mesh = plsc.VectorSubcoreMesh(core_axis_name="c", subcore_axis_name="s")
    kern = pl.kernel(
        _sc_id_body,
        mesh=mesh,
        compiler_params=pltpu.CompilerParams(use_tc_tiling_on_sc=True),
        out_type=[jax.ShapeDtypeStruct((B, NUM_ID * 128), jnp.float32)],
        scratch_types=[
            pltpu.VMEM((RPW,), jnp.int32),
            pltpu.VMEM((RPW, 128), jnp.float32),
        ],
    )
    return kern(emb128, q4_flat)[0]


def _sc_body(emb_det_h, emb_addf_h, fm_id_h, fm_det_h, fm_addf_h,
             idxT_h, ixdet_h, ixaddf_h,
             o_det, o_addf, o_fm1, o_det1, o_addf1,
             idx_v, buf32, buf16, q_v, val_v, acc_v, bag_v):
    f32 = jnp.float32
    i32 = jnp.int32
    wid = lax.axis_index("s") * 2 + lax.axis_index("c")
    rb = wid * RPW
    i16 = lax.iota(i32, 16)

    for i in range(RPW // 16):
        acc_v[pl.ds(i * 16, 16)] = jnp.zeros((16,), f32)
    @pl.loop(0, NUM_ID)
    def _(f):
        pltpu.sync_copy(idxT_h.at[f].at[pl.ds(rb, RPW)], idx_v.at[pl.ds(0, RPW)])
        for i in range(RPW // 16):
            q_v[pl.ds(i * 16, 16)] = jnp.right_shift(idx_v[pl.ds(i * 16, 16)], 4)
        pltpu.sync_copy(fm_id_h.at[f].at[q_v.at[pl.ds(0, RPW)]],
                        buf16.at[pl.ds(0, RPW)])
        for i in range(RPW // 16):
            lane = idx_v[pl.ds(i * 16, 16)] & 15
            g = plsc.load_gather(buf16, [i16 + i * 16, lane])
            acc_v[pl.ds(i * 16, 16)] = acc_v[pl.ds(i * 16, 16)] + g
    pltpu.sync_copy(acc_v.at[pl.ds(0, RPW)], o_fm1.at[pl.ds(rb, RPW)])

    def bag(emb_h, fm_h, ix_h, o_bag, o_1d):
        base = wid * PW_BAG
        pltpu.sync_copy(ix_h.at[pl.ds(base, PW_BAG)], idx_v.at[pl.ds(0, PW_BAG)])

        @pl.loop(0, NCH)
        def _(c):
            pltpu.sync_copy(emb_h.at[idx_v.at[pl.ds(c * CH_BAG, CH_BAG)]],
                            buf32.at[pl.ds(0, CH_BAG)])
            for i in range(CH_BAG // 16):
                q_v[pl.ds(i * 16, 16)] = jnp.right_shift(
                    idx_v[pl.ds(c * CH_BAG + i * 16, 16)], 4)
            pltpu.sync_copy(fm_h.at[q_v.at[pl.ds(0, CH_BAG)]],
                            buf16.at[pl.ds(0, CH_BAG)])

            @pl.loop(0, 16)
            def _(b):
                a0 = jnp.zeros((16,), f32)
                a1 = jnp.zeros((16,), f32)
                for r in range(SEQ_LEN):
                    row = b * SEQ_LEN + r
                    a0 = a0 + buf32[row, pl.ds(0, 16)]
                    a1 = a1 + buf32[row, pl.ds(16, 16)]
                bag_v[c * 16 + b, pl.ds(0, 16)] = a0
                bag_v[c * 16 + b, pl.ds(16, 16)] = a1

            for i in range(CH_BAG // 16):
                lane = idx_v[pl.ds(c * CH_BAG + i * 16, 16)] & 15
                val_v[pl.ds(i * 16, 16)] = plsc.load_gather(
                    buf16, [i16 + i * 16, lane])
            a = jnp.zeros((16,), f32)
            for r in range(SEQ_LEN):
                a = a + plsc.load_gather(val_v, [i16 * SEQ_LEN + r])
            acc_v[pl.ds(c * 16, 16)] = a
        pltpu.sync_copy(bag_v.at[pl.ds(0, RPW)], o_bag.at[pl.ds(rb, RPW)])
        pltpu.sync_copy(acc_v.at[pl.ds(0, RPW)], o_1d.at[pl.ds(rb, RPW)])

    bag(emb_det_h, fm_det_h, ixdet_h, o_det, o_det1)
    bag(emb_addf_h, fm_addf_h, ixaddf_h, o_addf, o_addf1)


def _sc_gather(emb_det, emb_addf, fm_id, fm_det, fm_addf,
               idxT, ix_det, ix_addf):
    mesh = plsc.VectorSubcoreMesh(core_axis_name="c", subcore_axis_name="s")
    f32 = jnp.float32
    kern = pl.kernel(
        _sc_body,
        mesh=mesh,
        compiler_params=pltpu.CompilerParams(use_tc_tiling_on_sc=False,
                                             needs_layout_passes=False),
        out_type=[
            jax.ShapeDtypeStruct((B, ED), f32),
            jax.ShapeDtypeStruct((B, ED), f32),
            jax.ShapeDtypeStruct((B,), f32),
            jax.ShapeDtypeStruct((B,), f32),
            jax.ShapeDtypeStruct((B,), f32),
        ],
        scratch_types=[
            pltpu.VMEM((PW_BAG,), jnp.int32),
            pltpu.VMEM((CH_BAG, ED), f32),
            pltpu.VMEM((CH_BAG, 16), f32),
            pltpu.VMEM((CH_BAG,), jnp.int32),
            pltpu.VMEM((CH_BAG,), f32),
            pltpu.VMEM((RPW,), f32),
            pltpu.VMEM((RPW, ED), f32),
        ],
    )
    return kern(emb_det, emb_addf, fm_id, fm_det, fm_addf,
                idxT, ix_det, ix_addf)


TILE = 256
NT = B // TILE


def _tc1_body(field4_r, oh4_r, dsum_r, asum_r, fm1_r, det1_r, addf1_r,
              dense_r, divd_r, diva_r, wlin_r, h0_r, s_r):
    field4 = field4_r[...]
    oh4 = oh4_r[...]
    divd = divd_r[...]
    diva = diva_r[...]

    cols = []
    for f in range(NUM_ID):
        v = jnp.zeros((TILE, ED), jnp.float32)
        for k in range(4):
            v = v + (field4[:, f * 128 + k * ED:f * 128 + (k + 1) * ED]
                     * oh4[:, f * 4 + k:f * 4 + k + 1])
        cols.append(v)
    field = jnp.concatenate(cols, axis=1)

    det_e = dsum_r[...] / divd
    addf_e = asum_r[...] / diva

    fsum = det_e + addf_e
    ssum = det_e * det_e + addf_e * addf_e
    for f in range(NUM_ID):
        v = cols[f]
        fsum = fsum + v
        ssum = ssum + v * v
    cross = 0.5 * jnp.sum(fsum * fsum - ssum, axis=1, keepdims=True)

    dense = dense_r[...]
    fmlin = jnp.sum(dense * wlin_r[...], axis=1, keepdims=True)
    s_r[...] = (cross + fmlin + fm1_r[...]
                + det1_r[...] / divd + addf1_r[...] / divd)

    pad = jnp.zeros((TILE, 1024 - (NUM_ID + 2) * ED - 23), jnp.float32)
    h0_r[...] = jnp.concatenate([field, det_e, addf_e, dense, pad], axis=1)


def _tc1(field4, oh4, dsum, asum, fm1, det1, addf1, dense, divd, diva, wlin_row):
    bs = lambda cols: pl.BlockSpec((TILE, cols), lambda i: (i, 0))
    full = lambda a: pl.BlockSpec(a.shape, lambda i: (0, 0))
    return pl.pallas_call(
        _tc1_body,
        grid=(NT,),
        in_specs=[bs(NUM_ID * 128), bs(NUM_ID * 4), bs(ED), bs(ED),
                  bs(1), bs(1), bs(1),
                  bs(23), bs(1), bs(1), full(wlin_row)],
        out_specs=[bs(1024), bs(1)],
        out_shape=[jax.ShapeDtypeStruct((B, 1024), jnp.float32),
                   jax.ShapeDtypeStruct((B, 1), jnp.float32)],
    )(field4, oh4, dsum, asum, fm1, det1, addf1, dense, divd, diva, wlin_row)


def _bn_cols(h, g, b):
    mu = jnp.mean(h, axis=0, keepdims=True)
    d = h - mu
    var = jnp.mean(d * d, axis=0, keepdims=True)
    return g * d * jax.lax.rsqrt(var + 1e-5) + b


def _tc2_body(h0_r, s_r, w1_r, b1_r, g1_r, be1_r, w2_r, b2_r, g2_r, be2_r,
              w3_r, b3_r, g3_r, be3_r, wd_r, bd_r, bng_r, bnb_r, out_r):
    f32 = jnp.float32
    h = jnp.dot(h0_r[...], w1_r[...], preferred_element_type=f32) + b1_r[...]
    h = jnp.maximum(_bn_cols(h, g1_r[...], be1_r[...]), 0.0)
    h = jnp.dot(h, w2_r[...], preferred_element_type=f32) + b2_r[...]
    h = jnp.maximum(_bn_cols(h, g2_r[...], be2_r[...]), 0.0)
    h = jnp.dot(h, w3_r[...], preferred_element_type=f32) + b3_r[...]
    h = jnp.maximum(_bn_cols(h, g3_r[...], be3_r[...]), 0.0)
    logit = jnp.dot(h, wd_r[...], preferred_element_type=f32) + bd_r[...]
    fm = _bn_cols(s_r[...], bng_r[...], bnb_r[...])
    out_r[...] = jax.nn.sigmoid(fm + logit)


def _tc2(h0, s, w1, b1, g1, be1, w2, b2, g2, be2, w3, b3, g3, be3, wd, bd, bng, bnb):
    args = (h0, s, w1, b1, g1, be1, w2, b2, g2, be2, w3, b3, g3, be3, wd, bd, bng, bnb)
    return pl.pallas_call(
        _tc2_body,
        out_shape=jax.ShapeDtypeStruct((B, 1), jnp.float32),
    )(*args)


def kernel(input_tensor, emb_id, emb_detail, emb_addf, fm1d_id, fm1d_detail,
           fm1d_addf, fm_lin_w, fm_bn_g, fm_bn_b, W1, b1, g1, be1, W2, b2, g2,
           be2, W3, b3, g3, be3, Wd, bd):
    x = input_tensor
    i32 = jnp.int32

    idx = x[:, :NUM_ID].astype(i32)
    det_ids = x[:, 28:78].astype(i32)
    addf_ids = x[:, 78:128].astype(i32)
    divd = x[:, 26:27]
    diva = x[:, 27:28]
    dense = x[:, 128:151]

    idxT = idx.T
    ix_det = det_ids.reshape(-1)
    ix_addf = addf_ids.reshape(-1)

    q4_flat = (idxT >> 2).reshape(-1)
    l4 = idx & 3
    oh4 = (l4[:, :, None] == jnp.arange(4, dtype=i32)).astype(
        jnp.float32).reshape(B, NUM_ID * 4)
    emb128 = _transpose_table(emb_id)
    field4 = _sc_id_gather(emb128, q4_flat)

    dsum, asum, fm1, det1, addf1 = _sc_gather(
        emb_detail, emb_addf,
        fm1d_id.reshape(NUM_ID, VOCAB // 16, 16),
        fm1d_detail.reshape(VOCAB // 16, 16),
        fm1d_addf.reshape(VOCAB // 16, 16),
        idxT, ix_det, ix_addf)

    h0, s = _tc1(field4, oh4, dsum, asum,
                 fm1.reshape(B, 1), det1.reshape(B, 1), addf1.reshape(B, 1),
                 dense, divd, diva, fm_lin_w.reshape(1, 23))

    w1p = jnp.pad(W1, ((0, 1024 - W1.shape[0]), (0, 0)))
    return _tc2(h0, s, w1p,
                b1.reshape(1, -1), g1.reshape(1, -1), be1.reshape(1, -1),
                W2, b2.reshape(1, -1), g2.reshape(1, -1), be2.reshape(1, -1),
                W3, b3.reshape(1, -1), g3.reshape(1, -1), be3.reshape(1, -1),
                Wd, bd.reshape(1, 1), fm_bn_g.reshape(1, 1), fm_bn_b.reshape(1, 1))


# speedup vs baseline: 1.4931x; 1.4931x over previous
import jax
import jax.numpy as jnp
from jax import lax
from jax.experimental import pallas as pl
from jax.experimental.pallas import tpu as pltpu
from jax.experimental.pallas import tpu_sc as plsc

B = 4096
NUM_ID = 26
SEQ_LEN = 50
VOCAB = 100000
ED = 32
NW = 32
RPW = B // NW
PW_BAG = RPW * SEQ_LEN
CH_BAG = 800
NCH = PW_BAG // CH_BAG


def _sc_body(emb_id_h, emb_det_h, emb_addf_h, fm_id_h, fm_det_h, fm_addf_h,
             idxT_h, ixdet_h, ixaddf_h,
             o_field, o_det, o_addf, o_fm1, o_det1, o_addf1,
             idx_v, buf32, buf16, q_v, val_v, acc_v, bag_v):
    f32 = jnp.float32
    i32 = jnp.int32
    wid = lax.axis_index("s") * 2 + lax.axis_index("c")
    rb = wid * RPW
    i16 = lax.iota(i32, 16)

    for i in range(RPW // 16):
        acc_v[pl.ds(i * 16, 16)] = jnp.zeros((16,), f32)
    @pl.loop(0, NUM_ID)
    def _(f):
        pltpu.sync_copy(idxT_h.at[f].at[pl.ds(rb, RPW)], idx_v.at[pl.ds(0, RPW)])
        pltpu.sync_copy(emb_id_h.at[f].at[idx_v.at[pl.ds(0, RPW)]],
                        buf32.at[pl.ds(0, RPW)])
        pltpu.sync_copy(buf32.at[pl.ds(0, RPW)],
                        o_field.at[pl.ds(rb, RPW), pl.ds(f * ED, ED)])
        for i in range(RPW // 16):
            q_v[pl.ds(i * 16, 16)] = jnp.right_shift(idx_v[pl.ds(i * 16, 16)], 4)
        pltpu.sync_copy(fm_id_h.at[f].at[q_v.at[pl.ds(0, RPW)]],
                        buf16.at[pl.ds(0, RPW)])
        for i in range(RPW // 16):
            lane = idx_v[pl.ds(i * 16, 16)] & 15
            g = plsc.load_gather(buf16, [i16 + i * 16, lane])
            acc_v[pl.ds(i * 16, 16)] = acc_v[pl.ds(i * 16, 16)] + g
    pltpu.sync_copy(acc_v.at[pl.ds(0, RPW)], o_fm1.at[pl.ds(rb, RPW)])

    def bag(emb_h, fm_h, ix_h, o_bag, o_1d):
        base = wid * PW_BAG
        pltpu.sync_copy(ix_h.at[pl.ds(base, PW_BAG)], idx_v.at[pl.ds(0, PW_BAG)])

        @pl.loop(0, NCH)
        def _(c):
            pltpu.sync_copy(emb_h.at[idx_v.at[pl.ds(c * CH_BAG, CH_BAG)]],
                            buf32.at[pl.ds(0, CH_BAG)])
            for i in range(CH_BAG // 16):
                q_v[pl.ds(i * 16, 16)] = jnp.right_shift(
                    idx_v[pl.ds(c * CH_BAG + i * 16, 16)], 4)
            pltpu.sync_copy(fm_h.at[q_v.at[pl.ds(0, CH_BAG)]],
                            buf16.at[pl.ds(0, CH_BAG)])

            @pl.loop(0, 16)
            def _(b):
                a0 = jnp.zeros((16,), f32)
                a1 = jnp.zeros((16,), f32)
                for r in range(SEQ_LEN):
                    row = b * SEQ_LEN + r
                    a0 = a0 + buf32[row, pl.ds(0, 16)]
                    a1 = a1 + buf32[row, pl.ds(16, 16)]
                bag_v[c * 16 + b, pl.ds(0, 16)] = a0
                bag_v[c * 16 + b, pl.ds(16, 16)] = a1

            for i in range(CH_BAG // 16):
                lane = idx_v[pl.ds(c * CH_BAG + i * 16, 16)] & 15
                val_v[pl.ds(i * 16, 16)] = plsc.load_gather(
                    buf16, [i16 + i * 16, lane])
            a = jnp.zeros((16,), f32)
            for r in range(SEQ_LEN):
                a = a + plsc.load_gather(val_v, [i16 * SEQ_LEN + r])
            acc_v[pl.ds(c * 16, 16)] = a
        pltpu.sync_copy(bag_v.at[pl.ds(0, RPW)], o_bag.at[pl.ds(rb, RPW)])
        pltpu.sync_copy(acc_v.at[pl.ds(0, RPW)], o_1d.at[pl.ds(rb, RPW)])

    bag(emb_det_h, fm_det_h, ixdet_h, o_det, o_det1)
    bag(emb_addf_h, fm_addf_h, ixaddf_h, o_addf, o_addf1)


def _sc_gather(emb_id, emb_det, emb_addf, fm_id, fm_det, fm_addf,
               idxT, ix_det, ix_addf):
    mesh = plsc.VectorSubcoreMesh(core_axis_name="c", subcore_axis_name="s")
    f32 = jnp.float32
    kern = pl.kernel(
        _sc_body,
        mesh=mesh,
        compiler_params=pltpu.CompilerParams(use_tc_tiling_on_sc=False,
                                             needs_layout_passes=False),
        out_type=[
            jax.ShapeDtypeStruct((B, NUM_ID * ED), f32),
            jax.ShapeDtypeStruct((B, ED), f32),
            jax.ShapeDtypeStruct((B, ED), f32),
            jax.ShapeDtypeStruct((B,), f32),
            jax.ShapeDtypeStruct((B,), f32),
            jax.ShapeDtypeStruct((B,), f32),
        ],
        scratch_types=[
            pltpu.VMEM((PW_BAG,), jnp.int32),
            pltpu.VMEM((CH_BAG, ED), f32),
            pltpu.VMEM((CH_BAG, 16), f32),
            pltpu.VMEM((CH_BAG,), jnp.int32),
            pltpu.VMEM((CH_BAG,), f32),
            pltpu.VMEM((RPW,), f32),
            pltpu.VMEM((RPW, ED), f32),
        ],
    )
    return kern(emb_id, emb_det, emb_addf, fm_id, fm_det, fm_addf,
                idxT, ix_det, ix_addf)


TILE = 256
NT = B // TILE


def _tc1_body(field_r, dsum_r, asum_r, fm1_r, det1_r, addf1_r,
              dense_r, divd_r, diva_r, wlin_r, h0_r, s_r):
    field = field_r[...]
    divd = divd_r[...]
    diva = diva_r[...]

    det_e = dsum_r[...] / divd
    addf_e = asum_r[...] / diva

    fsum = det_e + addf_e
    ssum = det_e * det_e + addf_e * addf_e
    for f in range(NUM_ID):
        v = field[:, f * ED:(f + 1) * ED]
        fsum = fsum + v
        ssum = ssum + v * v
    cross = 0.5 * jnp.sum(fsum * fsum - ssum, axis=1, keepdims=True)

    dense = dense_r[...]
    fmlin = jnp.sum(dense * wlin_r[...], axis=1, keepdims=True)
    s_r[...] = (cross + fmlin + fm1_r[...]
                + det1_r[...] / divd + addf1_r[...] / divd)

    pad = jnp.zeros((TILE, 1024 - (NUM_ID + 2) * ED - 23), jnp.float32)
    h0_r[...] = jnp.concatenate([field, det_e, addf_e, dense, pad], axis=1)


def _tc1(field, dsum, asum, fm1, det1, addf1, dense, divd, diva, wlin_row):
    bs = lambda cols: pl.BlockSpec((TILE, cols), lambda i: (i, 0))
    full = lambda a: pl.BlockSpec(a.shape, lambda i: (0, 0))
    return pl.pallas_call(
        _tc1_body,
        grid=(NT,),
        in_specs=[bs(NUM_ID * ED), bs(ED), bs(ED), bs(1), bs(1), bs(1),
                  bs(23), bs(1), bs(1), full(wlin_row)],
        out_specs=[bs(1024), bs(1)],
        out_shape=[jax.ShapeDtypeStruct((B, 1024), jnp.float32),
                   jax.ShapeDtypeStruct((B, 1), jnp.float32)],
    )(field, dsum, asum, fm1, det1, addf1, dense, divd, diva, wlin_row)


def _bn_cols(h, g, b):
    mu = jnp.mean(h, axis=0, keepdims=True)
    d = h - mu
    var = jnp.mean(d * d, axis=0, keepdims=True)
    return g * d * jax.lax.rsqrt(var + 1e-5) + b


def _tc2_body(h0_r, s_r, w1_r, b1_r, g1_r, be1_r, w2_r, b2_r, g2_r, be2_r,
              w3_r, b3_r, g3_r, be3_r, wd_r, bd_r, bng_r, bnb_r, out_r):
    f32 = jnp.float32
    h = jnp.dot(h0_r[...], w1_r[...], preferred_element_type=f32) + b1_r[...]
    h = jnp.maximum(_bn_cols(h, g1_r[...], be1_r[...]), 0.0)
    h = jnp.dot(h, w2_r[...], preferred_element_type=f32) + b2_r[...]
    h = jnp.maximum(_bn_cols(h, g2_r[...], be2_r[...]), 0.0)
    h = jnp.dot(h, w3_r[...], preferred_element_type=f32) + b3_r[...]
    h = jnp.maximum(_bn_cols(h, g3_r[...], be3_r[...]), 0.0)
    logit = jnp.dot(h, wd_r[...], preferred_element_type=f32) + bd_r[...]
    fm = _bn_cols(s_r[...], bng_r[...], bnb_r[...])
    out_r[...] = jax.nn.sigmoid(fm + logit)


def _tc2(h0, s, w1, b1, g1, be1, w2, b2, g2, be2, w3, b3, g3, be3, wd, bd, bng, bnb):
    args = (h0, s, w1, b1, g1, be1, w2, b2, g2, be2, w3, b3, g3, be3, wd, bd, bng, bnb)
    return pl.pallas_call(
        _tc2_body,
        out_shape=jax.ShapeDtypeStruct((B, 1), jnp.float32),
    )(*args)


def kernel(input_tensor, emb_id, emb_detail, emb_addf, fm1d_id, fm1d_detail,
           fm1d_addf, fm_lin_w, fm_bn_g, fm_bn_b, W1, b1, g1, be1, W2, b2, g2,
           be2, W3, b3, g3, be3, Wd, bd):
    x = input_tensor
    i32 = jnp.int32

    idx = x[:, :NUM_ID].astype(i32)
    det_ids = x[:, 28:78].astype(i32)
    addf_ids = x[:, 78:128].astype(i32)
    divd = x[:, 26:27]
    diva = x[:, 27:28]
    dense = x[:, 128:151]

    idxT = idx.T
    ix_det = det_ids.reshape(-1)
    ix_addf = addf_ids.reshape(-1)

    field, dsum, asum, fm1, det1, addf1 = _sc_gather(
        emb_id, emb_detail, emb_addf,
        fm1d_id.reshape(NUM_ID, VOCAB // 16, 16),
        fm1d_detail.reshape(VOCAB // 16, 16),
        fm1d_addf.reshape(VOCAB // 16, 16),
        idxT, ix_det, ix_addf)

    h0, s = _tc1(field, dsum, asum,
                 fm1.reshape(B, 1), det1.reshape(B, 1), addf1.reshape(B, 1),
                 dense, divd, diva, fm_lin_w.reshape(1, 23))

    w1p = jnp.pad(W1, ((0, 1024 - W1.shape[0]), (0, 0)))
    return _tc2(h0, s, w1p,
                b1.reshape(1, -1), g1.reshape(1, -1), be1.reshape(1, -1),
                W2, b2.reshape(1, -1), g2.reshape(1, -1), be2.reshape(1, -1),
                W3, b3.reshape(1, -1), g3.reshape(1, -1), be3.reshape(1, -1),
                Wd, bd.reshape(1, 1), fm_bn_g.reshape(1, 1), fm_bn_b.reshape(1, 1))
